# Initial kernel scaffold; baseline (speedup 1.0000x reference)
import jax
import jax.numpy as jnp
from jax.experimental import pallas as pl


def kernel(x, Wp, bp, codebook, Wo, bo):
    raise NotImplementedError("write your pallas kernel here")


# speedup vs baseline: 1.5622x; 1.5622x over previous
import functools

import jax
import jax.numpy as jnp
from jax.experimental import pallas as pl
from jax.experimental.pallas import tpu as pltpu

B, T, H = 4, 2048, 1024
G, V, D = 2, 320, 128
N = B * T
TN = 512
NT = N // TN


def _pq_kernel(x_ref, wpt_ref, bp_ref, cb_ref, wot_ref, bo_ref,
               q_ref, perp_ref, commit_ref, counts_ref, sse_ref):
    i = pl.program_id(0)

    @pl.when(i == 0)
    def _init():
        counts_ref[...] = jnp.zeros_like(counts_ref)
        sse_ref[0, 0] = 0.0

    x = x_ref[...]
    logits = jnp.dot(x, wpt_ref[...], preferred_element_type=jnp.float32)
    logits = logits + bp_ref[...]

    iota = jax.lax.broadcasted_iota(jnp.int32, (TN, V), 1)

    def onehot(l):
        m = jnp.max(l, axis=1, keepdims=True)
        first = jnp.min(jnp.where(l == m, iota, V), axis=1, keepdims=True)
        return (iota == first).astype(jnp.float32)

    oh0 = onehot(logits[:, :V])
    oh1 = onehot(logits[:, V:])
    counts_ref[0:1, :] = counts_ref[0:1, :] + jnp.sum(oh0, axis=0, keepdims=True)
    counts_ref[1:2, :] = counts_ref[1:2, :] + jnp.sum(oh1, axis=0, keepdims=True)

    g0 = jnp.dot(oh0, cb_ref[0], preferred_element_type=jnp.float32)
    g1 = jnp.dot(oh1, cb_ref[1], preferred_element_type=jnp.float32)
    gath = jnp.concatenate([g0, g1], axis=1)
    q = jnp.dot(gath, wot_ref[...], preferred_element_type=jnp.float32)
    q = q + bo_ref[...]
    q_ref[...] = q
    sse_ref[0, 0] += jnp.sum((x - q) ** 2)

    @pl.when(i == NT - 1)
    def _fin():
        avg = counts_ref[...] / N
        ent = -jnp.sum(avg * jnp.log(avg + 1e-9), axis=1, keepdims=True)
        perp_ref[...] = jnp.sum(jnp.exp(ent), axis=0, keepdims=True) / G
        commit_ref[...] = jnp.full((1, 1), sse_ref[0, 0] / (N * H), jnp.float32)


@functools.partial(jax.jit, static_argnames=("interpret",))
def kernel(x, Wp, bp, codebook, Wo, bo, interpret=False):
    x2d = x.reshape(N, H)
    wpt = Wp.T
    wot = Wo.T
    bp2d = bp.reshape(1, G * V)
    bo2d = bo.reshape(1, H)

    q2d, perp, commit = pl.pallas_call(
        _pq_kernel,
        grid=(NT,),
        in_specs=[
            pl.BlockSpec((TN, H), lambda i: (i, 0)),
            pl.BlockSpec((H, G * V), lambda i: (0, 0)),
            pl.BlockSpec((1, G * V), lambda i: (0, 0)),
            pl.BlockSpec((G, V, D), lambda i: (0, 0, 0)),
            pl.BlockSpec((G * D, H), lambda i: (0, 0)),
            pl.BlockSpec((1, H), lambda i: (0, 0)),
        ],
        out_specs=[
            pl.BlockSpec((TN, H), lambda i: (i, 0)),
            pl.BlockSpec((1, 1), lambda i: (0, 0)),
            pl.BlockSpec((1, 1), lambda i: (0, 0)),
        ],
        out_shape=[
            jax.ShapeDtypeStruct((N, H), jnp.float32),
            jax.ShapeDtypeStruct((1, 1), jnp.float32),
            jax.ShapeDtypeStruct((1, 1), jnp.float32),
        ],
        scratch_shapes=[
            pltpu.VMEM((G, V), jnp.float32),
            pltpu.SMEM((1, 1), jnp.float32),
        ],
        interpret=interpret,
    )(x2d, wpt, bp2d, codebook, wot, bo2d)

    return q2d.reshape(B, T, H), perp[0, 0], commit[0, 0]


# speedup vs baseline: 1.7130x; 1.0966x over previous
import functools

import jax
import jax.numpy as jnp
from jax.experimental import pallas as pl
from jax.experimental.pallas import tpu as pltpu

B, T, H = 4, 2048, 1024
G, V, D = 2, 320, 128
N = B * T
TN = 1024
NT = N // TN


def _pq_kernel(x_ref, wpt_ref, bp_ref, cb_ref, wot_ref, bo_ref,
               q_ref, perp_ref, commit_ref, counts_ref, sse_ref):
    i = pl.program_id(0)

    @pl.when(i == 0)
    def _init():
        counts_ref[...] = jnp.zeros_like(counts_ref)
        sse_ref[0, 0] = 0.0

    x = x_ref[...]
    logits = jnp.dot(x, wpt_ref[...], preferred_element_type=jnp.float32)
    logits = logits + bp_ref[...]

    iota = jax.lax.broadcasted_iota(jnp.int32, (TN, V), 1)

    def onehot(l):
        m = jnp.max(l, axis=1, keepdims=True)
        first = jnp.min(jnp.where(l == m, iota, V), axis=1, keepdims=True)
        return (iota == first).astype(jnp.float32)

    oh0 = onehot(logits[:, :V])
    oh1 = onehot(logits[:, V:])
    counts_ref[0:1, :] = counts_ref[0:1, :] + jnp.sum(oh0, axis=0, keepdims=True)
    counts_ref[1:2, :] = counts_ref[1:2, :] + jnp.sum(oh1, axis=0, keepdims=True)

    g0 = jnp.dot(oh0, cb_ref[0], preferred_element_type=jnp.float32)
    g1 = jnp.dot(oh1, cb_ref[1], preferred_element_type=jnp.float32)
    gath = jnp.concatenate([g0, g1], axis=1)
    q = jnp.dot(gath, wot_ref[...], preferred_element_type=jnp.float32)
    q = q + bo_ref[...]
    q_ref[...] = q
    sse_ref[0, 0] += jnp.sum((x - q) ** 2)

    @pl.when(i == NT - 1)
    def _fin():
        avg = counts_ref[...] / N
        ent = -jnp.sum(avg * jnp.log(avg + 1e-9), axis=1, keepdims=True)
        perp_ref[...] = jnp.sum(jnp.exp(ent), axis=0, keepdims=True) / G
        commit_ref[...] = jnp.full((1, 1), sse_ref[0, 0] / (N * H), jnp.float32)


@functools.partial(jax.jit, static_argnames=("interpret",))
def kernel(x, Wp, bp, codebook, Wo, bo, interpret=False):
    x2d = x.reshape(N, H)
    wpt = Wp.T
    wot = Wo.T
    bp2d = bp.reshape(1, G * V)
    bo2d = bo.reshape(1, H)

    q2d, perp, commit = pl.pallas_call(
        _pq_kernel,
        grid=(NT,),
        in_specs=[
            pl.BlockSpec((TN, H), lambda i: (i, 0)),
            pl.BlockSpec((H, G * V), lambda i: (0, 0)),
            pl.BlockSpec((1, G * V), lambda i: (0, 0)),
            pl.BlockSpec((G, V, D), lambda i: (0, 0, 0)),
            pl.BlockSpec((G * D, H), lambda i: (0, 0)),
            pl.BlockSpec((1, H), lambda i: (0, 0)),
        ],
        out_specs=[
            pl.BlockSpec((TN, H), lambda i: (i, 0)),
            pl.BlockSpec((1, 1), lambda i: (0, 0)),
            pl.BlockSpec((1, 1), lambda i: (0, 0)),
        ],
        out_shape=[
            jax.ShapeDtypeStruct((N, H), jnp.float32),
            jax.ShapeDtypeStruct((1, 1), jnp.float32),
            jax.ShapeDtypeStruct((1, 1), jnp.float32),
        ],
        scratch_shapes=[
            pltpu.VMEM((G, V), jnp.float32),
            pltpu.SMEM((1, 1), jnp.float32),
        ],
        interpret=interpret,
    )(x2d, wpt, bp2d, codebook, wot, bo2d)

    return q2d.reshape(B, T, H), perp[0, 0], commit[0, 0]


# speedup vs baseline: 1.7269x; 1.0081x over previous
import functools

import jax
import jax.numpy as jnp
from jax.experimental import pallas as pl
from jax.experimental.pallas import tpu as pltpu

B, T, H = 4, 2048, 1024
G, V, D = 2, 320, 128
N = B * T
TN = 2048
NT = N // TN


def _pq_kernel(x_ref, wpt_ref, bp_ref, cb_ref, wot_ref, bo_ref,
               q_ref, perp_ref, commit_ref, counts_ref, sse_ref):
    i = pl.program_id(0)

    @pl.when(i == 0)
    def _init():
        counts_ref[...] = jnp.zeros_like(counts_ref)
        sse_ref[0, 0] = 0.0

    x = x_ref[...]
    logits = jnp.dot(x, wpt_ref[...], preferred_element_type=jnp.float32)
    logits = logits + bp_ref[...]

    iota = jax.lax.broadcasted_iota(jnp.int32, (TN, V), 1)

    def onehot(l):
        m = jnp.max(l, axis=1, keepdims=True)
        first = jnp.min(jnp.where(l == m, iota, V), axis=1, keepdims=True)
        return (iota == first).astype(jnp.float32)

    oh0 = onehot(logits[:, :V])
    oh1 = onehot(logits[:, V:])
    counts_ref[0:1, :] = counts_ref[0:1, :] + jnp.sum(oh0, axis=0, keepdims=True)
    counts_ref[1:2, :] = counts_ref[1:2, :] + jnp.sum(oh1, axis=0, keepdims=True)

    g0 = jnp.dot(oh0, cb_ref[0], preferred_element_type=jnp.float32)
    g1 = jnp.dot(oh1, cb_ref[1], preferred_element_type=jnp.float32)
    gath = jnp.concatenate([g0, g1], axis=1)
    q = jnp.dot(gath, wot_ref[...], preferred_element_type=jnp.float32)
    q = q + bo_ref[...]
    q_ref[...] = q
    sse_ref[0, 0] += jnp.sum((x - q) ** 2)

    @pl.when(i == NT - 1)
    def _fin():
        avg = counts_ref[...] / N
        ent = -jnp.sum(avg * jnp.log(avg + 1e-9), axis=1, keepdims=True)
        perp_ref[...] = jnp.sum(jnp.exp(ent), axis=0, keepdims=True) / G
        commit_ref[...] = jnp.full((1, 1), sse_ref[0, 0] / (N * H), jnp.float32)


@functools.partial(jax.jit, static_argnames=("interpret",))
def kernel(x, Wp, bp, codebook, Wo, bo, interpret=False):
    x2d = x.reshape(N, H)
    wpt = Wp.T
    wot = Wo.T
    bp2d = bp.reshape(1, G * V)
    bo2d = bo.reshape(1, H)

    q2d, perp, commit = pl.pallas_call(
        _pq_kernel,
        grid=(NT,),
        in_specs=[
            pl.BlockSpec((TN, H), lambda i: (i, 0)),
            pl.BlockSpec((H, G * V), lambda i: (0, 0)),
            pl.BlockSpec((1, G * V), lambda i: (0, 0)),
            pl.BlockSpec((G, V, D), lambda i: (0, 0, 0)),
            pl.BlockSpec((G * D, H), lambda i: (0, 0)),
            pl.BlockSpec((1, H), lambda i: (0, 0)),
        ],
        out_specs=[
            pl.BlockSpec((TN, H), lambda i: (i, 0)),
            pl.BlockSpec((1, 1), lambda i: (0, 0)),
            pl.BlockSpec((1, 1), lambda i: (0, 0)),
        ],
        out_shape=[
            jax.ShapeDtypeStruct((N, H), jnp.float32),
            jax.ShapeDtypeStruct((1, 1), jnp.float32),
            jax.ShapeDtypeStruct((1, 1), jnp.float32),
        ],
        scratch_shapes=[
            pltpu.VMEM((G, V), jnp.float32),
            pltpu.SMEM((1, 1), jnp.float32),
        ],
        interpret=interpret,
    )(x2d, wpt, bp2d, codebook, wot, bo2d)

    return q2d.reshape(B, T, H), perp[0, 0], commit[0, 0]


# speedup vs baseline: 1.9729x; 1.1424x over previous
import functools

import jax
import jax.numpy as jnp
from jax.experimental import pallas as pl
from jax.experimental.pallas import tpu as pltpu

B, T, H = 4, 2048, 1024
G, V, D = 2, 320, 128
N = B * T
TN = 2048
NT = N // TN


def _pq_kernel(x_ref, wpt_ref, bp_ref, cb_ref, wot_ref, bo_ref,
               q_ref, perp_ref, commit_ref, counts_ref, sse_ref):
    i = pl.program_id(0)

    @pl.when(i == 0)
    def _init():
        counts_ref[...] = jnp.zeros_like(counts_ref)
        sse_ref[...] = jnp.zeros_like(sse_ref)

    x = x_ref[...]
    logits = jnp.dot(x, wpt_ref[...], preferred_element_type=jnp.float32)
    logits = logits + bp_ref[...]

    iota = jax.lax.broadcasted_iota(jnp.int32, (TN, V), 1)

    def onehot(l):
        m = jnp.max(l, axis=1, keepdims=True)
        first = jnp.min(jnp.where(l == m, iota, V), axis=1, keepdims=True)
        return (iota == first).astype(jnp.float32)

    oh0 = onehot(logits[:, :V])
    oh1 = onehot(logits[:, V:])
    counts_ref[0:1, :] = counts_ref[0:1, :] + jnp.sum(oh0, axis=0, keepdims=True)
    counts_ref[1:2, :] = counts_ref[1:2, :] + jnp.sum(oh1, axis=0, keepdims=True)

    g0 = jnp.dot(oh0.astype(jnp.bfloat16), cb_ref[0].astype(jnp.bfloat16),
                 preferred_element_type=jnp.float32)
    g1 = jnp.dot(oh1.astype(jnp.bfloat16), cb_ref[1].astype(jnp.bfloat16),
                 preferred_element_type=jnp.float32)
    gath = jnp.concatenate([g0, g1], axis=1).astype(jnp.bfloat16)
    q = jnp.dot(gath, wot_ref[...].astype(jnp.bfloat16),
                preferred_element_type=jnp.float32)
    q = q + bo_ref[...]
    q_ref[...] = q
    r = (x - q) ** 2
    sse_ref[...] = sse_ref[...] + jnp.sum(r, axis=0, keepdims=True)

    @pl.when(i == NT - 1)
    def _fin():
        avg = counts_ref[...] / N
        ent = -jnp.sum(avg * jnp.log(avg + 1e-9), axis=1, keepdims=True)
        perp_ref[...] = jnp.sum(jnp.exp(ent), axis=0, keepdims=True) / G
        commit_ref[...] = jnp.sum(sse_ref[...], axis=1, keepdims=True) / (N * H)


@functools.partial(jax.jit, static_argnames=("interpret",))
def kernel(x, Wp, bp, codebook, Wo, bo, interpret=False):
    x2d = x.reshape(N, H)
    wpt = Wp.T
    wot = Wo.T
    bp2d = bp.reshape(1, G * V)
    bo2d = bo.reshape(1, H)

    q2d, perp, commit = pl.pallas_call(
        _pq_kernel,
        grid=(NT,),
        in_specs=[
            pl.BlockSpec((TN, H), lambda i: (i, 0)),
            pl.BlockSpec((H, G * V), lambda i: (0, 0)),
            pl.BlockSpec((1, G * V), lambda i: (0, 0)),
            pl.BlockSpec((G, V, D), lambda i: (0, 0, 0)),
            pl.BlockSpec((G * D, H), lambda i: (0, 0)),
            pl.BlockSpec((1, H), lambda i: (0, 0)),
        ],
        out_specs=[
            pl.BlockSpec((TN, H), lambda i: (i, 0)),
            pl.BlockSpec((1, 1), lambda i: (0, 0)),
            pl.BlockSpec((1, 1), lambda i: (0, 0)),
        ],
        out_shape=[
            jax.ShapeDtypeStruct((N, H), jnp.float32),
            jax.ShapeDtypeStruct((1, 1), jnp.float32),
            jax.ShapeDtypeStruct((1, 1), jnp.float32),
        ],
        scratch_shapes=[
            pltpu.VMEM((G, V), jnp.float32),
            pltpu.VMEM((1, H), jnp.float32),
        ],
        interpret=interpret,
    )(x2d, wpt, bp2d, codebook, wot, bo2d)

    return q2d.reshape(B, T, H), perp[0, 0], commit[0, 0]
